# baseline (speedup 1.0000x reference)
import functools

import jax
import jax.numpy as jnp
from jax import lax
from jax.experimental import pallas as pl
from jax.experimental.pallas import tpu as pltpu
from jax.experimental.pallas import tpu_sc as plsc

N = 10000
D = 128
H = 128

NC = 2
NS = 16
NW = NC * NS

CH = 128
UP_CPW = 80
B_CPW = 8

NPAD = 10240
RPT = NPAD // NS
ACC_R = RPT + 8
SCAN = 2048
SROWS = SCAN // CH
FCAP = 2304
BATCH = 128


def _sc_segment_sums(xpad, battr_pad, up_src, up_dst, b_src, b_dst):
    i32 = jnp.int32
    mesh = plsc.VectorSubcoreMesh(
        core_axis_name="c", subcore_axis_name="s",
        num_cores=NC, num_subcores=NS)

    @functools.partial(
        pl.kernel,
        out_type=jax.ShapeDtypeStruct((2 * NC, NPAD, D), jnp.float32),
        mesh=mesh,
        compiler_params=pltpu.CompilerParams(needs_layout_passes=False),
        scratch_types=[
            pltpu.VMEM((ACC_R, D), jnp.float32),
            pltpu.VMEM((SROWS, CH), jnp.int32),
            pltpu.VMEM((SROWS, CH), jnp.int32),
            pltpu.VMEM((SROWS, CH), jnp.int32),
            pltpu.VMEM((SROWS, CH), jnp.int32),
            pltpu.VMEM((FCAP,), jnp.int32),
            pltpu.VMEM((BATCH, D), jnp.float32),
            pltpu.VMEM((BATCH, D), jnp.float32),
            pltpu.VMEM((BATCH,), jnp.int32),
            pltpu.VMEM((BATCH,), jnp.int32),
            pltpu.VMEM((BATCH + 16,), jnp.int32),
            pltpu.VMEM((BATCH + 16,), jnp.int32),
            pltpu.SemaphoreType.DMA,
            pltpu.SemaphoreType.DMA,
            pltpu.SemaphoreType.DMA,
            pltpu.SemaphoreType.DMA,
        ],
    )
    def k(xpad_hbm, battr_hbm, up_src_hbm, up_dst_hbm, b_src_hbm, b_dst_hbm,
          out_hbm, acc, sa_src, sa_dst, sb_src, sb_dst, fpak,
          rows0, rows1, st_s0, st_s1, st_d0, st_d1,
          ssem0, ssem1, gsem0, gsem1):
        c = lax.axis_index("c")
        s = lax.axis_index("s")
        lo = s * RPT

        scan_bufs = ((sa_src, sa_dst, ssem0), (sb_src, sb_dst, ssem1))
        rows = (rows0, rows1)
        st_s = (st_s0, st_s1)
        st_d = (st_d0, st_d1)
        gsem = (gsem0, gsem1)
        zvec = jnp.zeros((16,), jnp.float32)

        def zero_acc():
            def zrow(r, carry):
                for kk in range(D // 16):
                    acc[r, pl.ds(kk * 16, 16)] = zvec
                return carry
            lax.fori_loop(0, ACC_R, zrow, 0)

        def accum_batch(p, table_hbm):
            pltpu.make_async_copy(
                table_hbm.at[st_s[p]], rows[p], gsem[p]).wait()

            def arow(r, carry):
                m = st_d[p][pl.ds(r, 16)][0]
                for kk in range(D // 16):
                    plsc.addupdate(acc.at[m, pl.ds(kk * 16, 16)],
                                   rows[p][r, pl.ds(kk * 16, 16)])
                return carry
            lax.fori_loop(0, BATCH, arow, 0)

        def flush(off, bseq, table_hbm):
            for p in (0, 1):
                @pl.when(bseq % 2 == p)
                def _():
                    for kk in range(BATCH // 16):
                        sl = pl.ds(kk * 16, 16)
                        v = fpak[pl.ds(off + kk * 16, 16)]
                        st_s[p][sl] = v >> 10
                        st_d[p][sl] = v & 1023
                    pltpu.async_copy(table_hbm.at[st_s[p]], rows[p], gsem[p])

            @pl.when(bseq > 0)
            def _():
                for p in (0, 1):
                    @pl.when((bseq - 1) % 2 == p)
                    def _():
                        accum_batch(p, table_hbm)

        def scan_chunk(csrc, cdst, cnt):
            def vrow(rr, cnt):
                for kk in range(CH // 16):
                    sl = pl.ds(kk * 16, 16)
                    dl = cdst[rr, sl] - lo
                    mask = dl.astype(jnp.uint32) < jnp.uint32(RPT)
                    mi = jnp.where(mask, jnp.ones((16,), i32),
                                   jnp.zeros((16,), i32))
                    incl = plsc.cumsum(mi)
                    idx = incl - mi + cnt
                    packed = (csrc[rr, sl] << 10) | (dl & 1023)
                    plsc.store_scatter(fpak, [idx], packed, mask=mask)
                    cnt = cnt + incl[15]
                return cnt
            return lax.fori_loop(0, SROWS, vrow, cnt)

        def after_chunk(cnt, bseq, table_hbm):
            nb = cnt >> 7

            def fb(b, bseq):
                flush(b * BATCH, bseq, table_hbm)
                return bseq + 1
            bseq = lax.fori_loop(0, nb, fb, bseq)
            off = nb << 7
            for kk in range(BATCH // 16):
                fpak[pl.ds(kk * 16, 16)] = fpak[pl.ds(off + kk * 16, 16)]
            return cnt - off, bseq

        def run_phase(table_hbm, src_hbm, dst_hbm, hbase, nchunks, slot):
            zero_acc()

            def issue(i, bufset):
                bsrc, bdst, ssem = bufset
                pltpu.async_copy(
                    src_hbm.at[pl.ds(hbase + i * SROWS, SROWS)], bsrc, ssem)
                pltpu.async_copy(
                    dst_hbm.at[pl.ds(hbase + i * SROWS, SROWS)], bdst, ssem)

            issue(0, scan_bufs[0])
            issue(1, scan_bufs[1])
            npairs = nchunks // 2

            def pair(t, carry):
                cnt, bseq = carry
                for pi in (0, 1):
                    bsrc, bdst, ssem = scan_bufs[pi]
                    i = 2 * t + pi
                    pltpu.make_async_copy(
                        src_hbm.at[pl.ds(hbase + i * SROWS, SROWS)],
                        bsrc, ssem).wait()
                    pltpu.make_async_copy(
                        dst_hbm.at[pl.ds(hbase + i * SROWS, SROWS)],
                        bdst, ssem).wait()
                    cnt = scan_chunk(bsrc, bdst, cnt)
                    cnt, bseq = after_chunk(cnt, bseq, table_hbm)

                    @pl.when(t < npairs - 1)
                    def _():
                        issue(i + 2, scan_bufs[pi])
                return cnt, bseq
            cnt, bseq = lax.fori_loop(0, npairs, pair,
                                      (i32(0), i32(0)))

            padv = jnp.full((16,), (N << 10) | RPT, i32)
            for kk in range(BATCH // 16):
                fpak[pl.ds(cnt + kk * 16, 16)] = padv
            flush(0, bseq, table_hbm)
            bseq = bseq + 1

            for p in (0, 1):
                @pl.when((bseq - 1) % 2 == p)
                def _():
                    accum_batch(p, table_hbm)

            pltpu.sync_copy(acc.at[pl.ds(0, RPT)],
                            out_hbm.at[slot, pl.ds(lo, RPT)])

        up_half = UP_CPW * NW * CH // NC // SCAN
        b_half = B_CPW * NW * CH // NC // SCAN
        run_phase(xpad_hbm, up_src_hbm, up_dst_hbm,
                  c * (up_half * SROWS), up_half, c)
        run_phase(battr_hbm, b_src_hbm, b_dst_hbm,
                  c * (b_half * SROWS), b_half, NC + c)

    return k(xpad, battr_pad, up_src, up_dst, b_src, b_dst)


def _bn_relu(h, gamma, beta):
    m = jnp.mean(h, axis=0, keepdims=True)
    v = jnp.mean((h - m) ** 2, axis=0, keepdims=True)
    return jnp.maximum(gamma * (h - m) / jnp.sqrt(v + 1e-5) + beta, 0.0)


def _dense_body(x_ref, parts_ref,
                W1u_ref, b1u_ref, g1u_ref, be1u_ref,
                W2u_ref, b2u_ref, g2u_ref, be2u_ref,
                W1b_ref, b1b_ref, g1b_ref, be1b_ref,
                W2b_ref, b2b_ref, g2b_ref, be2b_ref,
                Wcu_ref, Wcb_ref, bc_ref, gc_ref, bec_ref, eps_ref, o_ref):
    xv = x_ref[...]
    scale = 1.0 + eps_ref[0, 0]
    agg_up = parts_ref[0, :N] + parts_ref[1, :N]
    agg_b = parts_ref[2, :N] + parts_ref[3, :N]

    def mlp(h, W1, b1, g1, be1, W2, b2, g2, be2):
        h = _bn_relu(jnp.dot(h, W1, preferred_element_type=jnp.float32) + b1,
                     g1, be1)
        h = _bn_relu(jnp.dot(h, W2, preferred_element_type=jnp.float32) + b2,
                     g2, be2)
        return h

    out_up = mlp(agg_up + scale * xv,
                 W1u_ref[...], b1u_ref[...], g1u_ref[...], be1u_ref[...],
                 W2u_ref[...], b2u_ref[...], g2u_ref[...], be2u_ref[...])
    out_b = mlp(agg_b + scale * xv,
                W1b_ref[...], b1b_ref[...], g1b_ref[...], be1b_ref[...],
                W2b_ref[...], b2b_ref[...], g2b_ref[...], be2b_ref[...])
    catw = (jnp.dot(out_up, Wcu_ref[...], preferred_element_type=jnp.float32)
            + jnp.dot(out_b, Wcb_ref[...], preferred_element_type=jnp.float32)
            + bc_ref[...])
    o_ref[...] = _bn_relu(catw, gc_ref[...], bec_ref[...])


def _pad_idx(idx, total, fill=None):
    pad = total - idx.shape[0]
    if fill is None:
        tail = jnp.arange(pad, dtype=jnp.int32) % NPAD
    else:
        tail = jnp.full((pad,), fill, jnp.int32)
    idx = jnp.concatenate([idx, tail])
    return idx.reshape(-1, CH)


def kernel(x, up_index, up_attr, boundary_attr, boundary_index,
           W1u, b1u, g1u, be1u, W2u, b2u, g2u, be2u,
           W1b, b1b, g1b, be1b, W2b, b2b, g2b, be2b,
           Wc, bc, gc, bec, eps1):
    zrow8 = jnp.zeros((8, D), jnp.float32)
    xpad = jnp.concatenate([x, zrow8], axis=0)
    battr_pad = jnp.concatenate([boundary_attr, zrow8], axis=0)

    up_src = _pad_idx(up_index[0], UP_CPW * NW * CH, N)
    up_dst = _pad_idx(up_index[1], UP_CPW * NW * CH)
    b_src = _pad_idx(boundary_index[0], B_CPW * NW * CH, N)
    b_dst = _pad_idx(boundary_index[1], B_CPW * NW * CH)

    parts = _sc_segment_sums(xpad, battr_pad, up_src, up_dst, b_src, b_dst)

    row = lambda a: a.reshape(1, -1)
    return pl.pallas_call(
        _dense_body,
        out_shape=jax.ShapeDtypeStruct((N, H), jnp.float32),
        compiler_params=pltpu.CompilerParams(
            vmem_limit_bytes=120 * 1024 * 1024),
    )(x, parts,
      W1u, row(b1u), row(g1u), row(be1u),
      W2u, row(b2u), row(g2u), row(be2u),
      W1b, row(b1b), row(g1b), row(be1b),
      W2b, row(b2b), row(g2b), row(be2b),
      Wc[:H], Wc[H:], row(bc), row(gc), row(bec),
      eps1.reshape(1, 1))


# speedup vs baseline: 1.1593x; 1.1593x over previous
import functools

import jax
import jax.numpy as jnp
from jax import lax
from jax.experimental import pallas as pl
from jax.experimental.pallas import tpu as pltpu
from jax.experimental.pallas import tpu_sc as plsc

N = 10000
D = 128
H = 128

NC = 2
NS = 16
NW = NC * NS

CH = 128
UP_CPW = 80
B_CPW = 8

NPAD = 10240
RPT = NPAD // NS
ACC_R = RPT + 8
SCAN = 2048
SROWS = SCAN // CH
FCAP = 2304
BATCH = 128


def _sc_segment_sums(xpad, battr_pad, up_src, up_dst, b_src, b_dst):
    i32 = jnp.int32
    mesh = plsc.VectorSubcoreMesh(
        core_axis_name="c", subcore_axis_name="s",
        num_cores=NC, num_subcores=NS)

    @functools.partial(
        pl.kernel,
        out_type=jax.ShapeDtypeStruct((2 * NC, NPAD, D), jnp.float32),
        mesh=mesh,
        compiler_params=pltpu.CompilerParams(needs_layout_passes=False),
        scratch_types=[
            pltpu.VMEM((ACC_R, D), jnp.float32),
            pltpu.VMEM((SROWS, CH), jnp.int32),
            pltpu.VMEM((SROWS, CH), jnp.int32),
            pltpu.VMEM((SROWS, CH), jnp.int32),
            pltpu.VMEM((SROWS, CH), jnp.int32),
            pltpu.VMEM((FCAP,), jnp.int32),
            pltpu.VMEM((BATCH, D), jnp.float32),
            pltpu.VMEM((BATCH, D), jnp.float32),
            pltpu.VMEM((BATCH,), jnp.int32),
            pltpu.VMEM((BATCH,), jnp.int32),
            pltpu.VMEM((BATCH + 16,), jnp.int32),
            pltpu.VMEM((BATCH + 16,), jnp.int32),
            pltpu.SemaphoreType.DMA,
            pltpu.SemaphoreType.DMA,
            pltpu.SemaphoreType.DMA,
            pltpu.SemaphoreType.DMA,
        ],
    )
    def k(xpad_hbm, battr_hbm, up_src_hbm, up_dst_hbm, b_src_hbm, b_dst_hbm,
          out_hbm, acc, sa_src, sa_dst, sb_src, sb_dst, fpak,
          rows0, rows1, st_s0, st_s1, st_d0, st_d1,
          ssem0, ssem1, gsem0, gsem1):
        c = lax.axis_index("c")
        s = lax.axis_index("s")
        lo = s * RPT

        scan_bufs = ((sa_src, sa_dst, ssem0), (sb_src, sb_dst, ssem1))
        rows = (rows0, rows1)
        st_s = (st_s0, st_s1)
        st_d = (st_d0, st_d1)
        gsem = (gsem0, gsem1)
        zvec = jnp.zeros((16,), jnp.float32)

        def zero_acc():
            def zrow(r, carry):
                for kk in range(D // 16):
                    acc[r, pl.ds(kk * 16, 16)] = zvec
                return carry
            lax.fori_loop(0, ACC_R, zrow, 0)

        def accum_batch(p, table_hbm):
            pltpu.make_async_copy(
                table_hbm.at[st_s[p]], rows[p], gsem[p]).wait()

            def agrp(g, carry):
                mv = st_d[p][pl.ds(g * 16, 16)]
                for l in range(16):
                    m = mv[l]
                    r = g * 16 + l
                    for kk in range(D // 16):
                        sl = pl.ds(kk * 16, 16)
                        plsc.addupdate(acc.at[m, sl], rows[p][r, sl])
                return carry
            lax.fori_loop(0, BATCH // 16, agrp, 0)

        def flush(off, bseq, table_hbm):
            for p in (0, 1):
                @pl.when(bseq % 2 == p)
                def _():
                    for kk in range(BATCH // 16):
                        sl = pl.ds(kk * 16, 16)
                        v = fpak[pl.ds(off + kk * 16, 16)]
                        st_s[p][sl] = v >> 10
                        st_d[p][sl] = v & 1023
                    pltpu.async_copy(table_hbm.at[st_s[p]], rows[p], gsem[p])

            @pl.when(bseq > 0)
            def _():
                for p in (0, 1):
                    @pl.when((bseq - 1) % 2 == p)
                    def _():
                        accum_batch(p, table_hbm)

        one16 = jnp.ones((16,), i32)
        zero16 = jnp.zeros((16,), i32)

        def scan_chunk(csrc, cdst, cnt):
            def vrow(rr, cnt):
                subs = []
                for kk in range(CH // 16):
                    sl = pl.ds(kk * 16, 16)
                    dl = cdst[rr, sl] - lo
                    mask = dl.astype(jnp.uint32) < jnp.uint32(RPT)
                    mi = jnp.where(mask, one16, zero16)
                    incl = plsc.cumsum(mi)
                    packed = (csrc[rr, sl] << 10) | (dl & 1023)
                    subs.append((mask, mi, incl, packed))
                for mask, mi, incl, packed in subs:
                    plsc.store_scatter(fpak, [incl - mi + cnt], packed,
                                       mask=mask)
                    cnt = cnt + incl[15]
                return cnt
            return lax.fori_loop(0, SROWS, vrow, cnt)

        def after_chunk(cnt, bseq, table_hbm):
            nb = cnt >> 7

            def fb(b, bseq):
                flush(b * BATCH, bseq, table_hbm)
                return bseq + 1
            bseq = lax.fori_loop(0, nb, fb, bseq)
            off = nb << 7
            for kk in range(BATCH // 16):
                fpak[pl.ds(kk * 16, 16)] = fpak[pl.ds(off + kk * 16, 16)]
            return cnt - off, bseq

        def run_phase(table_hbm, src_hbm, dst_hbm, hbase, nchunks, slot):
            zero_acc()

            def issue(i, bufset):
                bsrc, bdst, ssem = bufset
                pltpu.async_copy(
                    src_hbm.at[pl.ds(hbase + i * SROWS, SROWS)], bsrc, ssem)
                pltpu.async_copy(
                    dst_hbm.at[pl.ds(hbase + i * SROWS, SROWS)], bdst, ssem)

            issue(0, scan_bufs[0])
            issue(1, scan_bufs[1])
            npairs = nchunks // 2

            def pair(t, carry):
                cnt, bseq = carry
                for pi in (0, 1):
                    bsrc, bdst, ssem = scan_bufs[pi]
                    i = 2 * t + pi
                    pltpu.make_async_copy(
                        src_hbm.at[pl.ds(hbase + i * SROWS, SROWS)],
                        bsrc, ssem).wait()
                    pltpu.make_async_copy(
                        dst_hbm.at[pl.ds(hbase + i * SROWS, SROWS)],
                        bdst, ssem).wait()
                    cnt = scan_chunk(bsrc, bdst, cnt)
                    cnt, bseq = after_chunk(cnt, bseq, table_hbm)

                    @pl.when(t < npairs - 1)
                    def _():
                        issue(i + 2, scan_bufs[pi])
                return cnt, bseq
            cnt, bseq = lax.fori_loop(0, npairs, pair,
                                      (i32(0), i32(0)))

            padv = jnp.full((16,), (N << 10) | RPT, i32)
            for kk in range(BATCH // 16):
                fpak[pl.ds(cnt + kk * 16, 16)] = padv
            flush(0, bseq, table_hbm)
            bseq = bseq + 1

            for p in (0, 1):
                @pl.when((bseq - 1) % 2 == p)
                def _():
                    accum_batch(p, table_hbm)

            pltpu.sync_copy(acc.at[pl.ds(0, RPT)],
                            out_hbm.at[slot, pl.ds(lo, RPT)])

        up_half = UP_CPW * NW * CH // NC // SCAN
        b_half = B_CPW * NW * CH // NC // SCAN
        run_phase(xpad_hbm, up_src_hbm, up_dst_hbm,
                  c * (up_half * SROWS), up_half, c)
        run_phase(battr_hbm, b_src_hbm, b_dst_hbm,
                  c * (b_half * SROWS), b_half, NC + c)

    return k(xpad, battr_pad, up_src, up_dst, b_src, b_dst)


def _bn_relu(h, gamma, beta):
    m = jnp.mean(h, axis=0, keepdims=True)
    v = jnp.mean((h - m) ** 2, axis=0, keepdims=True)
    return jnp.maximum(gamma * (h - m) / jnp.sqrt(v + 1e-5) + beta, 0.0)


def _dense_body(x_ref, parts_ref,
                W1u_ref, b1u_ref, g1u_ref, be1u_ref,
                W2u_ref, b2u_ref, g2u_ref, be2u_ref,
                W1b_ref, b1b_ref, g1b_ref, be1b_ref,
                W2b_ref, b2b_ref, g2b_ref, be2b_ref,
                Wcu_ref, Wcb_ref, bc_ref, gc_ref, bec_ref, eps_ref, o_ref):
    xv = x_ref[...]
    scale = 1.0 + eps_ref[0, 0]
    agg_up = parts_ref[0, :N] + parts_ref[1, :N]
    agg_b = parts_ref[2, :N] + parts_ref[3, :N]

    def mlp(h, W1, b1, g1, be1, W2, b2, g2, be2):
        h = _bn_relu(jnp.dot(h, W1, preferred_element_type=jnp.float32) + b1,
                     g1, be1)
        h = _bn_relu(jnp.dot(h, W2, preferred_element_type=jnp.float32) + b2,
                     g2, be2)
        return h

    out_up = mlp(agg_up + scale * xv,
                 W1u_ref[...], b1u_ref[...], g1u_ref[...], be1u_ref[...],
                 W2u_ref[...], b2u_ref[...], g2u_ref[...], be2u_ref[...])
    out_b = mlp(agg_b + scale * xv,
                W1b_ref[...], b1b_ref[...], g1b_ref[...], be1b_ref[...],
                W2b_ref[...], b2b_ref[...], g2b_ref[...], be2b_ref[...])
    catw = (jnp.dot(out_up, Wcu_ref[...], preferred_element_type=jnp.float32)
            + jnp.dot(out_b, Wcb_ref[...], preferred_element_type=jnp.float32)
            + bc_ref[...])
    o_ref[...] = _bn_relu(catw, gc_ref[...], bec_ref[...])


def _pad_idx(idx, total, fill=None):
    pad = total - idx.shape[0]
    if fill is None:
        tail = jnp.arange(pad, dtype=jnp.int32) % NPAD
    else:
        tail = jnp.full((pad,), fill, jnp.int32)
    idx = jnp.concatenate([idx, tail])
    return idx.reshape(-1, CH)


def kernel(x, up_index, up_attr, boundary_attr, boundary_index,
           W1u, b1u, g1u, be1u, W2u, b2u, g2u, be2u,
           W1b, b1b, g1b, be1b, W2b, b2b, g2b, be2b,
           Wc, bc, gc, bec, eps1):
    zrow8 = jnp.zeros((8, D), jnp.float32)
    xpad = jnp.concatenate([x, zrow8], axis=0)
    battr_pad = jnp.concatenate([boundary_attr, zrow8], axis=0)

    up_src = _pad_idx(up_index[0], UP_CPW * NW * CH, N)
    up_dst = _pad_idx(up_index[1], UP_CPW * NW * CH)
    b_src = _pad_idx(boundary_index[0], B_CPW * NW * CH, N)
    b_dst = _pad_idx(boundary_index[1], B_CPW * NW * CH)

    parts = _sc_segment_sums(xpad, battr_pad, up_src, up_dst, b_src, b_dst)

    row = lambda a: a.reshape(1, -1)
    return pl.pallas_call(
        _dense_body,
        out_shape=jax.ShapeDtypeStruct((N, H), jnp.float32),
        compiler_params=pltpu.CompilerParams(
            vmem_limit_bytes=120 * 1024 * 1024),
    )(x, parts,
      W1u, row(b1u), row(g1u), row(be1u),
      W2u, row(b2u), row(g2u), row(be2u),
      W1b, row(b1b), row(g1b), row(be1b),
      W2b, row(b2b), row(g2b), row(be2b),
      Wc[:H], Wc[H:], row(bc), row(gc), row(bec),
      eps1.reshape(1, 1))


# speedup vs baseline: 7.0147x; 6.0510x over previous
import functools

import jax
import jax.numpy as jnp
from jax import lax
from jax.experimental import pallas as pl
from jax.experimental.pallas import tpu as pltpu
from jax.experimental.pallas import tpu_sc as plsc

N = 10000
D = 128
H = 128

NC = 2
NS = 16
NW = NC * NS

ZPAD = 64
CH = 128
UP_CPW = 80
B_CPW = 8

NPAD = 10240
ROWS_PER_TILE = NPAD // NS
ZROWS = 16
IDX_STAGE = 40


def _sc_segment_sums(xpad, battr_pad, up_src, up_dst, b_src, b_dst):
    mesh = plsc.VectorSubcoreMesh(
        core_axis_name="c", subcore_axis_name="s",
        num_cores=NC, num_subcores=NS)

    @functools.partial(
        pl.kernel,
        out_type=jax.ShapeDtypeStruct((2 * NC, NPAD, D), jnp.float32),
        mesh=mesh,
        scratch_types=[
            pltpu.VMEM_SHARED((NPAD, D), jnp.float32),
            pltpu.VMEM((IDX_STAGE, CH), jnp.int32),
            pltpu.VMEM((IDX_STAGE, CH), jnp.int32),
            pltpu.VMEM((CH, D), jnp.float32),
            pltpu.VMEM((CH, D), jnp.float32),
            pltpu.VMEM((ZROWS, D), jnp.float32),
            pltpu.SemaphoreType.DMA,
            pltpu.SemaphoreType.DMA,
            pltpu.SemaphoreType.DMA,
        ],
    )
    def k(xpad_hbm, battr_hbm, up_src_hbm, up_dst_hbm, b_src_hbm, b_dst_hbm,
          out_hbm, acc, src_idx, dst_idx, rows0, rows1, zbuf, sem0, sem1,
          zsem):
        c = lax.axis_index("c")
        s = lax.axis_index("s")
        w = c * NS + s
        row0 = s * ROWS_PER_TILE

        def zrow(r, carry):
            for cc in range(D // 16):
                zbuf[r, pl.ds(cc * 16, 16)] = jnp.zeros((16,), jnp.float32)
            return carry
        lax.fori_loop(0, ZROWS, zrow, 0)

        def zero_acc():
            nz = ROWS_PER_TILE // ZROWS

            def zissue(kk, carry):
                pltpu.async_copy(
                    zbuf, acc.at[pl.ds(row0 + kk * ZROWS, ZROWS)], zsem)
                return carry
            lax.fori_loop(0, nz, zissue, 0)

            def zdrain(kk, carry):
                pltpu.make_async_copy(
                    zbuf, acc.at[pl.ds(row0 + kk * ZROWS, ZROWS)], zsem).wait()
                return carry
            lax.fori_loop(0, nz, zdrain, 0)

        bufs = (rows0, rows1)
        sems = (sem0, sem1)

        def run_phase(table_hbm, src_hbm, dst_hbm, cpw):
            nstages = (cpw + IDX_STAGE - 1) // IDX_STAGE
            for st in range(nstages):
                sc = min(IDX_STAGE, cpw - st * IDX_STAGE)
                base = w * cpw + st * IDX_STAGE
                pltpu.sync_copy(src_hbm.at[pl.ds(base, sc)],
                                src_idx.at[pl.ds(0, sc)])
                pltpu.sync_copy(dst_hbm.at[pl.ds(base, sc)],
                                dst_idx.at[pl.ds(0, sc)])
                pltpu.async_copy(table_hbm.at[src_idx.at[0]], rows0, sem0)
                pltpu.async_copy(table_hbm.at[src_idx.at[1]], rows1, sem1)
                npairs = sc // 2

                def body(t, carry):
                    for b in range(2):
                        j = 2 * t + b
                        pltpu.make_async_copy(
                            table_hbm.at[src_idx.at[j]], bufs[b],
                            sems[b]).wait()
                        pltpu.sync_copy(bufs[b], acc.at[dst_idx.at[j]],
                                        add=True)

                        @pl.when(t < npairs - 1)
                        def _():
                            pltpu.async_copy(
                                table_hbm.at[src_idx.at[j + 2]], bufs[b],
                                sems[b])
                    return carry
                lax.fori_loop(0, npairs, body, 0)

        def writeback(slot):
            pltpu.sync_copy(acc.at[pl.ds(row0, ROWS_PER_TILE)],
                            out_hbm.at[slot, pl.ds(row0, ROWS_PER_TILE)])

        zero_acc()
        plsc.subcore_barrier()
        run_phase(xpad_hbm, up_src_hbm, up_dst_hbm, UP_CPW)
        plsc.subcore_barrier()
        writeback(c)
        zero_acc()
        plsc.subcore_barrier()
        run_phase(battr_hbm, b_src_hbm, b_dst_hbm, B_CPW)
        plsc.subcore_barrier()
        writeback(NC + c)

    return k(xpad, battr_pad, up_src, up_dst, b_src, b_dst)


def _bn_relu(h, gamma, beta):
    m = jnp.mean(h, axis=0, keepdims=True)
    v = jnp.mean((h - m) ** 2, axis=0, keepdims=True)
    return jnp.maximum(gamma * (h - m) / jnp.sqrt(v + 1e-5) + beta, 0.0)


def _dense_body(x_ref, parts_ref,
                W1u_ref, b1u_ref, g1u_ref, be1u_ref,
                W2u_ref, b2u_ref, g2u_ref, be2u_ref,
                W1b_ref, b1b_ref, g1b_ref, be1b_ref,
                W2b_ref, b2b_ref, g2b_ref, be2b_ref,
                Wcu_ref, Wcb_ref, bc_ref, gc_ref, bec_ref, eps_ref, o_ref):
    xv = x_ref[...]
    scale = 1.0 + eps_ref[0, 0]
    agg_up = parts_ref[0, :N] + parts_ref[1, :N]
    agg_b = parts_ref[2, :N] + parts_ref[3, :N]

    def mlp(h, W1, b1, g1, be1, W2, b2, g2, be2):
        h = _bn_relu(jnp.dot(h, W1, preferred_element_type=jnp.float32) + b1,
                     g1, be1)
        h = _bn_relu(jnp.dot(h, W2, preferred_element_type=jnp.float32) + b2,
                     g2, be2)
        return h

    out_up = mlp(agg_up + scale * xv,
                 W1u_ref[...], b1u_ref[...], g1u_ref[...], be1u_ref[...],
                 W2u_ref[...], b2u_ref[...], g2u_ref[...], be2u_ref[...])
    out_b = mlp(agg_b + scale * xv,
                W1b_ref[...], b1b_ref[...], g1b_ref[...], be1b_ref[...],
                W2b_ref[...], b2b_ref[...], g2b_ref[...], be2b_ref[...])
    catw = (jnp.dot(out_up, Wcu_ref[...], preferred_element_type=jnp.float32)
            + jnp.dot(out_b, Wcb_ref[...], preferred_element_type=jnp.float32)
            + bc_ref[...])
    o_ref[...] = _bn_relu(catw, gc_ref[...], bec_ref[...])


def _pad_idx(idx, total, zero_rows=False):
    pad = total - idx.shape[0]
    ar = jnp.arange(pad, dtype=jnp.int32)
    if zero_rows:
        tail = N + ar % ZPAD
    else:
        tail = ar % NPAD
    idx = jnp.concatenate([idx, tail])
    return idx.reshape(-1, CH)


def kernel(x, up_index, up_attr, boundary_attr, boundary_index,
           W1u, b1u, g1u, be1u, W2u, b2u, g2u, be2u,
           W1b, b1b, g1b, be1b, W2b, b2b, g2b, be2b,
           Wc, bc, gc, bec, eps1):
    zrows = jnp.zeros((ZPAD, D), jnp.float32)
    xpad = jnp.concatenate([x, zrows], axis=0)
    battr_pad = jnp.concatenate([boundary_attr, zrows], axis=0)

    up_src = _pad_idx(up_index[0], UP_CPW * NW * CH, zero_rows=True)
    up_dst = _pad_idx(up_index[1], UP_CPW * NW * CH)
    b_src = _pad_idx(boundary_index[0], B_CPW * NW * CH, zero_rows=True)
    b_dst = _pad_idx(boundary_index[1], B_CPW * NW * CH)

    parts = _sc_segment_sums(xpad, battr_pad, up_src, up_dst, b_src, b_dst)

    row = lambda a: a.reshape(1, -1)
    return pl.pallas_call(
        _dense_body,
        out_shape=jax.ShapeDtypeStruct((N, H), jnp.float32),
        compiler_params=pltpu.CompilerParams(
            vmem_limit_bytes=120 * 1024 * 1024),
    )(x, parts,
      W1u, row(b1u), row(g1u), row(be1u),
      W2u, row(b2u), row(g2u), row(be2u),
      W1b, row(b1b), row(g1b), row(be1b),
      W2b, row(b2b), row(g2b), row(be2b),
      Wc[:H], Wc[H:], row(bc), row(gc), row(bec),
      eps1.reshape(1, 1))


# speedup vs baseline: 7.5783x; 1.0803x over previous
import functools

import jax
import jax.numpy as jnp
from jax import lax
from jax.experimental import pallas as pl
from jax.experimental.pallas import tpu as pltpu
from jax.experimental.pallas import tpu_sc as plsc

N = 10000
D = 128
H = 128

NC = 2
NS = 16
NW = NC * NS

CH = 128
UP_CPW = 80
B_CPW = 8

NPAD = 10240
ROWS_PER_TILE = NPAD // NS
ZROWS = 16
IDX_STAGE = 40


def _sc_segment_sums(xpad, battr_pad, up_src, up_dst, b_src, b_dst):
    mesh = plsc.VectorSubcoreMesh(
        core_axis_name="c", subcore_axis_name="s",
        num_cores=NC, num_subcores=NS)

    @functools.partial(
        pl.kernel,
        out_type=jax.ShapeDtypeStruct((2 * NC, NPAD, D), jnp.float32),
        mesh=mesh,
        scratch_types=[
            pltpu.VMEM_SHARED((NPAD, D), jnp.float32),
            pltpu.VMEM((IDX_STAGE, CH), jnp.int32),
            pltpu.VMEM((IDX_STAGE, CH), jnp.int32),
            pltpu.VMEM((CH, D), jnp.float32),
            pltpu.VMEM((CH, D), jnp.float32),
            pltpu.VMEM((ZROWS, D), jnp.float32),
            pltpu.SemaphoreType.DMA,
            pltpu.SemaphoreType.DMA,
            pltpu.SemaphoreType.DMA,
        ],
    )
    def k(xpad_hbm, battr_hbm, up_src_hbm, up_dst_hbm, b_src_hbm, b_dst_hbm,
          out_hbm, acc, src_idx, dst_idx, rows0, rows1, zbuf, sem0, sem1,
          zsem):
        c = lax.axis_index("c")
        s = lax.axis_index("s")
        w = c * NS + s
        row0 = s * ROWS_PER_TILE

        def zrow(r, carry):
            for cc in range(D // 16):
                zbuf[r, pl.ds(cc * 16, 16)] = jnp.zeros((16,), jnp.float32)
            return carry
        lax.fori_loop(0, ZROWS, zrow, 0)

        def zero_acc():
            nz = ROWS_PER_TILE // ZROWS

            def zissue(kk, carry):
                pltpu.async_copy(
                    zbuf, acc.at[pl.ds(row0 + kk * ZROWS, ZROWS)], zsem)
                return carry
            lax.fori_loop(0, nz, zissue, 0)

            def zdrain(kk, carry):
                pltpu.make_async_copy(
                    zbuf, acc.at[pl.ds(row0 + kk * ZROWS, ZROWS)], zsem).wait()
                return carry
            lax.fori_loop(0, nz, zdrain, 0)

        bufs = (rows0, rows1)
        sems = (sem0, sem1)

        def run_phase(table_hbm, src_hbm, dst_hbm, cpw):
            nstages = (cpw + IDX_STAGE - 1) // IDX_STAGE
            for st in range(nstages):
                sc = min(IDX_STAGE, cpw - st * IDX_STAGE)
                base = w * cpw + st * IDX_STAGE
                pltpu.sync_copy(src_hbm.at[pl.ds(base, sc)],
                                src_idx.at[pl.ds(0, sc)])
                pltpu.sync_copy(dst_hbm.at[pl.ds(base, sc)],
                                dst_idx.at[pl.ds(0, sc)])
                pltpu.async_copy(table_hbm.at[src_idx.at[0]], rows0, sem0)
                pltpu.async_copy(table_hbm.at[src_idx.at[1]], rows1, sem1)
                npairs = sc // 2

                def body(t, carry):
                    for b in range(2):
                        j = 2 * t + b
                        pltpu.make_async_copy(
                            table_hbm.at[src_idx.at[j]], bufs[b],
                            sems[b]).wait()
                        pltpu.sync_copy(bufs[b], acc.at[dst_idx.at[j]],
                                        add=True)

                        @pl.when(t < npairs - 1)
                        def _():
                            pltpu.async_copy(
                                table_hbm.at[src_idx.at[j + 2]], bufs[b],
                                sems[b])
                    return carry
                lax.fori_loop(0, npairs, body, 0)

        def writeback(slot):
            pltpu.sync_copy(acc.at[pl.ds(row0, ROWS_PER_TILE)],
                            out_hbm.at[slot, pl.ds(row0, ROWS_PER_TILE)])

        zero_acc()
        plsc.subcore_barrier()
        run_phase(xpad_hbm, up_src_hbm, up_dst_hbm, UP_CPW)
        plsc.subcore_barrier()
        writeback(c)
        zero_acc()
        plsc.subcore_barrier()
        run_phase(battr_hbm, b_src_hbm, b_dst_hbm, B_CPW)
        plsc.subcore_barrier()
        writeback(NC + c)

    return k(xpad, battr_pad, up_src, up_dst, b_src, b_dst)


def _bn_relu(h, gamma, beta):
    m = jnp.mean(h, axis=0, keepdims=True)
    v = jnp.mean((h - m) ** 2, axis=0, keepdims=True)
    return jnp.maximum(gamma * (h - m) / jnp.sqrt(v + 1e-5) + beta, 0.0)


def _dense_body(x_ref, parts_ref,
                W1u_ref, b1u_ref, g1u_ref, be1u_ref,
                W2u_ref, b2u_ref, g2u_ref, be2u_ref,
                W1b_ref, b1b_ref, g1b_ref, be1b_ref,
                W2b_ref, b2b_ref, g2b_ref, be2b_ref,
                Wcu_ref, Wcb_ref, bc_ref, gc_ref, bec_ref, eps_ref, o_ref):
    xv = x_ref[...]
    scale = 1.0 + eps_ref[0, 0]
    agg_up = parts_ref[0, :N] + parts_ref[1, :N]
    agg_b = parts_ref[2, :N] + parts_ref[3, :N]

    def mlp(h, W1, b1, g1, be1, W2, b2, g2, be2):
        h = _bn_relu(jnp.dot(h, W1, preferred_element_type=jnp.float32) + b1,
                     g1, be1)
        h = _bn_relu(jnp.dot(h, W2, preferred_element_type=jnp.float32) + b2,
                     g2, be2)
        return h

    out_up = mlp(agg_up + scale * xv,
                 W1u_ref[...], b1u_ref[...], g1u_ref[...], be1u_ref[...],
                 W2u_ref[...], b2u_ref[...], g2u_ref[...], be2u_ref[...])
    out_b = mlp(agg_b + scale * xv,
                W1b_ref[...], b1b_ref[...], g1b_ref[...], be1b_ref[...],
                W2b_ref[...], b2b_ref[...], g2b_ref[...], be2b_ref[...])
    catw = (jnp.dot(out_up, Wcu_ref[...], preferred_element_type=jnp.float32)
            + jnp.dot(out_b, Wcb_ref[...], preferred_element_type=jnp.float32)
            + bc_ref[...])
    o_ref[...] = _bn_relu(catw, gc_ref[...], bec_ref[...])


def _pad_idx(idx, total, srcs=False):
    pad = total - idx.shape[0]
    ar = jnp.arange(pad, dtype=jnp.int32)
    if srcs:
        tail = ar % N
    else:
        tail = N + ar % (NPAD - N)
    idx = jnp.concatenate([idx, tail])
    return idx.reshape(-1, CH)


def kernel(x, up_index, up_attr, boundary_attr, boundary_index,
           W1u, b1u, g1u, be1u, W2u, b2u, g2u, be2u,
           W1b, b1b, g1b, be1b, W2b, b2b, g2b, be2b,
           Wc, bc, gc, bec, eps1):
    up_src = _pad_idx(up_index[0], UP_CPW * NW * CH, srcs=True)
    up_dst = _pad_idx(up_index[1], UP_CPW * NW * CH)
    b_src = _pad_idx(boundary_index[0], B_CPW * NW * CH, srcs=True)
    b_dst = _pad_idx(boundary_index[1], B_CPW * NW * CH)

    parts = _sc_segment_sums(x, boundary_attr, up_src, up_dst, b_src, b_dst)

    row = lambda a: a.reshape(1, -1)
    return pl.pallas_call(
        _dense_body,
        out_shape=jax.ShapeDtypeStruct((N, H), jnp.float32),
        compiler_params=pltpu.CompilerParams(
            vmem_limit_bytes=120 * 1024 * 1024),
    )(x, parts,
      W1u, row(b1u), row(g1u), row(be1u),
      W2u, row(b2u), row(g2u), row(be2u),
      W1b, row(b1b), row(g1b), row(be1b),
      W2b, row(b2b), row(g2b), row(be2b),
      Wc[:H], Wc[H:], row(bc), row(gc), row(bec),
      eps1.reshape(1, 1))
